# baseline (speedup 1.0000x reference)
import jax
import jax.numpy as jnp
from jax.experimental import pallas as pl
from jax.experimental.pallas import tpu as pltpu


def _diff_body(cu_ref, out_ref):
    def body(i, carry):
        out_ref[i] = cu_ref[i + 1] - cu_ref[i]
        return carry

    jax.lax.fori_loop(0, out_ref.shape[0], body, 0)


def kernel(values, cu_seqlens):
    n = cu_seqlens.shape[0] - 1
    row_lengths = pl.pallas_call(
        _diff_body,
        in_specs=[pl.BlockSpec(memory_space=pltpu.SMEM)],
        out_specs=pl.BlockSpec(memory_space=pltpu.SMEM),
        out_shape=jax.ShapeDtypeStruct((n,), cu_seqlens.dtype),
    )(cu_seqlens)
    return values, row_lengths
